# Initial kernel scaffold; baseline (speedup 1.0000x reference)
import jax
import jax.numpy as jnp
from jax.experimental import pallas as pl


def kernel(n1_x, n1_edge_index, n1_batch, n2_x, n2_edge_index, n2_batch, W_n1c1, b_n1c1, g_n1l1, be_n1l1, W_n1c2, b_n1c2, g_n1l2, be_n1l2, W_n2c1, b_n2c1, g_n2l1, be_n2l1, W_n2c2, b_n2c2, g_n2l2, be_n2l2, W_lin1, b_lin1, W_lin2, b_lin2):
    raise NotImplementedError("write your pallas kernel here")


# speedup vs baseline: 6.4810x; 6.4810x over previous
import functools

import jax
import jax.numpy as jnp
from jax import lax
from jax.experimental import pallas as pl
from jax.experimental.pallas import tpu as pltpu
from jax.experimental.pallas import tpu_sc as plsc

N = 10000
E = 320000
FEAT = 128
HID = 256
G = 16
CLS = 10

NP = 10240
EP = 327680
ROWS_PER_TILE = EP // 128 // 16
NPT = NP // 16
F32 = jnp.float32

_mesh = plsc.VectorSubcoreMesh(
    core_axis_name="c", subcore_axis_name="s", num_cores=2, num_subcores=16)


@functools.partial(
    pl.kernel,
    out_type=jax.ShapeDtypeStruct((2 * NP, 16), F32),
    mesh=_mesh,
    scratch_types=[
        pltpu.VMEM((16, 128), jnp.int32),
        pltpu.VMEM((128, 16), F32),
        pltpu.VMEM_SHARED((NP, 16), F32),
    ],
)
def _deg_kernel(dst_idx_hbm, zdeg_hbm, e0_hbm, deg_out, didx_v, e0_v, acc_sh):
    cid = lax.axis_index("c")
    sid = lax.axis_index("s")
    pltpu.sync_copy(zdeg_hbm.at[pl.ds(sid * NPT, NPT)],
                    acc_sh.at[pl.ds(sid * NPT, NPT)])
    pltpu.sync_copy(e0_hbm, e0_v)
    plsc.subcore_barrier()
    base = cid * (EP // 128) + sid * ROWS_PER_TILE

    def stage(ch, carry):
        pltpu.sync_copy(dst_idx_hbm.at[pl.ds(base + ch * 16, 16)], didx_v)
        for j in range(16):
            pltpu.sync_copy(e0_v, acc_sh.at[didx_v.at[j]], add=True)
        return carry

    lax.fori_loop(0, ROWS_PER_TILE // 16, stage, 0)
    plsc.subcore_barrier()
    pltpu.sync_copy(acc_sh.at[pl.ds(sid * NPT, NPT)],
                    deg_out.at[pl.ds(cid * NP + sid * NPT, NPT)])


@functools.partial(
    pl.kernel,
    out_type=jax.ShapeDtypeStruct((4 * NP, 128), F32),
    mesh=_mesh,
    scratch_types=[
        pltpu.VMEM((16, 128), jnp.int32),
        pltpu.VMEM((16, 128), jnp.int32),
        pltpu.VMEM((128, 128), F32),
        pltpu.VMEM_SHARED((NP, 128), F32),
    ],
)
def _agg_kernel(hp_hbm, src_idx_hbm, dst_idx_hbm, zrow_hbm, out_hbm,
                sidx_v, didx_v, rows_v, acc_sh):
    cid = lax.axis_index("c")
    sid = lax.axis_index("s")
    for g in range(2):
        pltpu.sync_copy(zrow_hbm.at[pl.ds(sid * NPT, NPT)],
                        acc_sh.at[pl.ds(sid * NPT, NPT)])
        plsc.subcore_barrier()
        sbase = (cid * 2 + g) * (EP // 128) + sid * ROWS_PER_TILE
        dbase = g * (EP // 128) + sid * ROWS_PER_TILE

        def stage(ch, carry):
            pltpu.sync_copy(src_idx_hbm.at[pl.ds(sbase + ch * 16, 16)], sidx_v)
            pltpu.sync_copy(dst_idx_hbm.at[pl.ds(dbase + ch * 16, 16)], didx_v)
            for j in range(16):
                pltpu.sync_copy(hp_hbm.at[sidx_v.at[j]], rows_v)
                pltpu.sync_copy(rows_v, acc_sh.at[didx_v.at[j]], add=True)
            return carry

        lax.fori_loop(0, ROWS_PER_TILE // 16, stage, 0)
        plsc.subcore_barrier()
        ob = (2 * g + cid) * NP + sid * NPT
        pltpu.sync_copy(acc_sh.at[pl.ds(sid * NPT, NPT)],
                        out_hbm.at[pl.ds(ob, NPT)])


BN = 1024


def _tca_body(x_ref, w_ref, deg_ref, out_ref):
    x = x_ref[0]
    w = w_ref[0]
    cnt = deg_ref[0][:, :1]
    dis = lax.rsqrt(cnt + 1.0)
    h = jnp.dot(x, w, preferred_element_type=F32) * dis
    out_ref[0, 0] = h[:, :128]
    out_ref[0, 1] = h[:, 128:]


def _ln_relu(pre, gam, bet):
    mu = jnp.mean(pre, axis=-1, keepdims=True)
    var = jnp.mean((pre - mu) * (pre - mu), axis=-1, keepdims=True)
    return jnp.maximum((pre - mu) * lax.rsqrt(var + 1e-5) * gam + bet, 0.0)


def _tcb_body(agg_ref, hp_ref, deg_ref, w2_ref, b1_ref, g1_ref, be1_ref,
              out_ref):
    a = agg_ref[0]
    acc = jnp.concatenate([a[0], a[1]], axis=-1)
    hv = hp_ref[0]
    hp = jnp.concatenate([hv[0], hv[1]], axis=-1)
    cnt = deg_ref[0][:, :1]
    dis = lax.rsqrt(cnt + 1.0)
    pre = dis * (acc + hp) + b1_ref[0, 0]
    z = _ln_relu(pre, g1_ref[0, 0], be1_ref[0, 0])
    h2 = jnp.dot(z, w2_ref[0], preferred_element_type=F32) * dis
    out_ref[0, 0] = h2[:, :128]
    out_ref[0, 1] = h2[:, 128:]


def _tcc_body(agg_ref, hp_ref, deg_ref, b2_ref, g2_ref, be2_ref, bat_ref,
              pool_ref, cnt_ref):
    a = agg_ref[0]
    acc = jnp.concatenate([a[0], a[1]], axis=-1)
    hv = hp_ref[0]
    hp = jnp.concatenate([hv[0], hv[1]], axis=-1)
    cnt = deg_ref[0][:, :1]
    dis = lax.rsqrt(cnt + 1.0)
    pre = dis * (acc + hp) + b2_ref[0, 0]
    z = _ln_relu(pre, g2_ref[0, 0], be2_ref[0, 0])
    bat = bat_ref[0, 0]
    oh = (bat[:, None] == lax.broadcasted_iota(jnp.int32, (BN, G), 1)
          ).astype(F32)
    pp = lax.dot_general(oh, z, (((0,), (0,)), ((), ())),
                         preferred_element_type=F32)
    cc = jnp.sum(oh, axis=0)
    i = pl.program_id(1)

    @pl.when(i == 0)
    def _():
        pool_ref[0] = jnp.zeros((G, HID), F32)
        cnt_ref[0] = jnp.zeros((G, 128), F32)

    pool_ref[0] += pp
    cnt_ref[0] += cc[:, None]


def _tcd_body(pool_ref, cnt_ref, w1_ref, b1_ref, w2_ref, b2_ref, out_ref):
    c1 = jnp.maximum(cnt_ref[0][:, :1], 1.0)
    c2 = jnp.maximum(cnt_ref[1][:, :1], 1.0)
    p = jnp.concatenate([pool_ref[0] / c1, pool_ref[1] / c2], axis=-1)
    y = jnp.maximum(
        jnp.dot(p, w1_ref[...], preferred_element_type=F32) + b1_ref[0], 0.0)
    out_ref[...] = jnp.dot(y, w2_ref[...],
                           preferred_element_type=F32) + b2_ref[0]


def kernel(n1_x, n1_edge_index, n1_batch, n2_x, n2_edge_index, n2_batch,
           W_n1c1, b_n1c1, g_n1l1, be_n1l1, W_n1c2, b_n1c2, g_n1l2, be_n1l2,
           W_n2c1, b_n2c1, g_n2l1, be_n2l1, W_n2c2, b_n2c2, g_n2l2, be_n2l2,
           W_lin1, b_lin1, W_lin2, b_lin2):
    xs = jnp.pad(jnp.stack([n1_x, n2_x]), ((0, 0), (0, NP - N), (0, 0)))
    W1s = jnp.stack([W_n1c1, W_n2c1])
    W2s = jnp.stack([W_n1c2, W_n2c2])
    b1s = jnp.stack([b_n1c1, b_n2c1]).reshape(2, 1, HID)
    g1s = jnp.stack([g_n1l1, g_n2l1]).reshape(2, 1, HID)
    be1s = jnp.stack([be_n1l1, be_n2l1]).reshape(2, 1, HID)
    b2s = jnp.stack([b_n1c2, b_n2c2]).reshape(2, 1, HID)
    g2s = jnp.stack([g_n1l2, g_n2l2]).reshape(2, 1, HID)
    be2s = jnp.stack([be_n1l2, be_n2l2]).reshape(2, 1, HID)
    batp = jnp.pad(jnp.stack([n1_batch, n2_batch]), ((0, 0), (0, NP - N)),
                   constant_values=G).reshape(2, 1, NP)
    srcs = jnp.stack([n1_edge_index[0], n2_edge_index[0]])
    dsts = jnp.stack([n1_edge_index[1], n2_edge_index[1]])
    srcp = jnp.pad(srcs, ((0, 0), (0, EP - E)), constant_values=N)
    dstp = jnp.pad(dsts, ((0, 0), (0, EP - E)), constant_values=N)
    offs = jnp.array([[0, 2], [1, 3]], jnp.int32) * NP
    src_idx = (srcp[None, :, :] + offs[:, :, None]).reshape(4 * EP // 128, 128)
    dst_idx = dstp.reshape(2 * EP // 128, 128)
    zrow = jnp.zeros((NP, 128), F32)
    zdeg = jnp.zeros((NP, 16), F32)
    e0 = jnp.zeros((128, 16), F32).at[:, 0].set(1.0)

    degs = _deg_kernel(dst_idx, zdeg, e0).reshape(2, NP, 16)

    hp1 = pl.pallas_call(
        _tca_body,
        grid=(2, NP // BN),
        in_specs=[
            pl.BlockSpec((1, BN, FEAT), lambda g, i: (g, i, 0)),
            pl.BlockSpec((1, FEAT, HID), lambda g, i: (g, 0, 0)),
            pl.BlockSpec((1, BN, 16), lambda g, i: (g, i, 0)),
        ],
        out_specs=pl.BlockSpec((1, 2, BN, 128), lambda g, i: (g, 0, i, 0)),
        out_shape=jax.ShapeDtypeStruct((2, 2, NP, 128), F32),
    )(xs, W1s, degs)

    agg1 = _agg_kernel(hp1.reshape(4 * NP, 128), src_idx, dst_idx,
                       zrow).reshape(2, 2, NP, 128)

    hp2 = pl.pallas_call(
        _tcb_body,
        grid=(2, NP // BN),
        in_specs=[
            pl.BlockSpec((1, 2, BN, 128), lambda g, i: (g, 0, i, 0)),
            pl.BlockSpec((1, 2, BN, 128), lambda g, i: (g, 0, i, 0)),
            pl.BlockSpec((1, BN, 16), lambda g, i: (g, i, 0)),
            pl.BlockSpec((1, HID, HID), lambda g, i: (g, 0, 0)),
            pl.BlockSpec((1, 1, HID), lambda g, i: (g, 0, 0)),
            pl.BlockSpec((1, 1, HID), lambda g, i: (g, 0, 0)),
            pl.BlockSpec((1, 1, HID), lambda g, i: (g, 0, 0)),
        ],
        out_specs=pl.BlockSpec((1, 2, BN, 128), lambda g, i: (g, 0, i, 0)),
        out_shape=jax.ShapeDtypeStruct((2, 2, NP, 128), F32),
    )(agg1, hp1, degs, W2s, b1s, g1s, be1s)

    agg2 = _agg_kernel(hp2.reshape(4 * NP, 128), src_idx, dst_idx,
                       zrow).reshape(2, 2, NP, 128)

    pool, cntb = pl.pallas_call(
        _tcc_body,
        grid=(2, NP // BN),
        in_specs=[
            pl.BlockSpec((1, 2, BN, 128), lambda g, i: (g, 0, i, 0)),
            pl.BlockSpec((1, 2, BN, 128), lambda g, i: (g, 0, i, 0)),
            pl.BlockSpec((1, BN, 16), lambda g, i: (g, i, 0)),
            pl.BlockSpec((1, 1, HID), lambda g, i: (g, 0, 0)),
            pl.BlockSpec((1, 1, HID), lambda g, i: (g, 0, 0)),
            pl.BlockSpec((1, 1, HID), lambda g, i: (g, 0, 0)),
            pl.BlockSpec((1, 1, BN), lambda g, i: (g, 0, i)),
        ],
        out_specs=[
            pl.BlockSpec((1, G, HID), lambda g, i: (g, 0, 0)),
            pl.BlockSpec((1, G, 128), lambda g, i: (g, 0, 0)),
        ],
        out_shape=[
            jax.ShapeDtypeStruct((2, G, HID), F32),
            jax.ShapeDtypeStruct((2, G, 128), F32),
        ],
    )(agg2, hp2, degs, b2s, g2s, be2s, batp)

    out = pl.pallas_call(
        _tcd_body,
        out_shape=jax.ShapeDtypeStruct((G, CLS), F32),
    )(pool, cntb, W_lin1, b_lin1.reshape(1, 128), W_lin2,
      b_lin2.reshape(1, CLS))
    return out


# speedup vs baseline: 7.6760x; 1.1844x over previous
import functools

import jax
import jax.numpy as jnp
from jax import lax
from jax.experimental import pallas as pl
from jax.experimental.pallas import tpu as pltpu
from jax.experimental.pallas import tpu_sc as plsc

N = 10000
E = 320000
FEAT = 128
HID = 256
G = 16
CLS = 10

NP = 10240
EP = 327680
ROWS_PER_TILE = EP // 128 // 16
NPT = NP // 16
F32 = jnp.float32

_mesh = plsc.VectorSubcoreMesh(
    core_axis_name="c", subcore_axis_name="s", num_cores=2, num_subcores=16)


@functools.partial(
    pl.kernel,
    out_type=jax.ShapeDtypeStruct((2 * NP, 16), F32),
    mesh=_mesh,
    scratch_types=[
        pltpu.VMEM((16, 128), jnp.int32),
        pltpu.VMEM((128, 16), F32),
        pltpu.VMEM_SHARED((NP, 16), F32),
    ],
)
def _deg_kernel(dst_idx_hbm, zdeg_hbm, e0_hbm, deg_out, didx_v, e0_v, acc_sh):
    cid = lax.axis_index("c")
    sid = lax.axis_index("s")
    pltpu.sync_copy(zdeg_hbm.at[pl.ds(sid * NPT, NPT)],
                    acc_sh.at[pl.ds(sid * NPT, NPT)])
    pltpu.sync_copy(e0_hbm, e0_v)
    plsc.subcore_barrier()
    base = cid * (EP // 128) + sid * ROWS_PER_TILE

    def stage(ch, carry):
        pltpu.sync_copy(dst_idx_hbm.at[pl.ds(base + ch * 16, 16)], didx_v)
        for j in range(16):
            pltpu.sync_copy(e0_v, acc_sh.at[didx_v.at[j]], add=True)
        return carry

    lax.fori_loop(0, ROWS_PER_TILE // 16, stage, 0)
    plsc.subcore_barrier()
    pltpu.sync_copy(acc_sh.at[pl.ds(sid * NPT, NPT)],
                    deg_out.at[pl.ds(cid * NP + sid * NPT, NPT)])


NBUF = 2
LAG = 1


@functools.partial(
    pl.kernel,
    out_type=jax.ShapeDtypeStruct((4 * NP, 128), F32),
    mesh=_mesh,
    scratch_types=[
        pltpu.VMEM((16, 128), jnp.int32),
        pltpu.VMEM((16, 128), jnp.int32),
        pltpu.VMEM((NBUF, 128, 128), F32),
    ]
    + [pltpu.SemaphoreType.DMA] * (2 * NBUF)
    + [pltpu.VMEM_SHARED((NP, 128), F32)],
)
def _agg_kernel(hp_hbm, src_idx_hbm, dst_idx_hbm, zrow_hbm, out_hbm,
                sidx_v, didx_v, rows_v, *rest):
    gsem = rest[:NBUF]
    ssem = rest[NBUF:2 * NBUF]
    acc_sh = rest[2 * NBUF]
    cid = lax.axis_index("c")
    sid = lax.axis_index("s")
    for g in range(2):
        pltpu.sync_copy(zrow_hbm.at[pl.ds(sid * NPT, NPT)],
                        acc_sh.at[pl.ds(sid * NPT, NPT)])
        plsc.subcore_barrier()
        sbase = (cid * 2 + g) * (EP // 128) + sid * ROWS_PER_TILE
        dbase = g * (EP // 128) + sid * ROWS_PER_TILE

        def stage(ch, carry):
            pltpu.sync_copy(src_idx_hbm.at[pl.ds(sbase + ch * 16, 16)], sidx_v)
            pltpu.sync_copy(dst_idx_hbm.at[pl.ds(dbase + ch * 16, 16)], didx_v)
            gd = [None] * NBUF
            sd = [None] * NBUF
            for j in range(16 + LAG):
                if j < 16:
                    b = j % NBUF
                    if sd[b] is not None:
                        sd[b].wait()
                    gd[b] = pltpu.async_copy(
                        hp_hbm.at[sidx_v.at[j]], rows_v.at[b], gsem[b])
                i = j - LAG
                if i >= 0:
                    bi = i % NBUF
                    gd[bi].wait()
                    sd[bi] = pltpu.async_copy(
                        rows_v.at[bi], acc_sh.at[didx_v.at[i]], ssem[bi],
                        add=True)
            for b in range(NBUF):
                sd[b].wait()
            return carry

        lax.fori_loop(0, ROWS_PER_TILE // 16, stage, 0)
        plsc.subcore_barrier()
        ob = (2 * g + cid) * NP + sid * NPT
        pltpu.sync_copy(acc_sh.at[pl.ds(sid * NPT, NPT)],
                        out_hbm.at[pl.ds(ob, NPT)])


BN = 1024


def _tca_body(x_ref, w_ref, deg_ref, out_ref):
    x = x_ref[0]
    w = w_ref[0]
    cnt = deg_ref[0][:, :1]
    dis = lax.rsqrt(cnt + 1.0)
    h = jnp.dot(x, w, preferred_element_type=F32) * dis
    out_ref[0, 0] = h[:, :128]
    out_ref[0, 1] = h[:, 128:]


def _ln_relu(pre, gam, bet):
    mu = jnp.mean(pre, axis=-1, keepdims=True)
    var = jnp.mean((pre - mu) * (pre - mu), axis=-1, keepdims=True)
    return jnp.maximum((pre - mu) * lax.rsqrt(var + 1e-5) * gam + bet, 0.0)


def _tcb_body(agg_ref, hp_ref, deg_ref, w2_ref, b1_ref, g1_ref, be1_ref,
              out_ref):
    a = agg_ref[0]
    acc = jnp.concatenate([a[0], a[1]], axis=-1)
    hv = hp_ref[0]
    hp = jnp.concatenate([hv[0], hv[1]], axis=-1)
    cnt = deg_ref[0][:, :1]
    dis = lax.rsqrt(cnt + 1.0)
    pre = dis * (acc + hp) + b1_ref[0, 0]
    z = _ln_relu(pre, g1_ref[0, 0], be1_ref[0, 0])
    h2 = jnp.dot(z, w2_ref[0], preferred_element_type=F32) * dis
    out_ref[0, 0] = h2[:, :128]
    out_ref[0, 1] = h2[:, 128:]


def _tcc_body(agg_ref, hp_ref, deg_ref, b2_ref, g2_ref, be2_ref, bat_ref,
              pool_ref, cnt_ref):
    a = agg_ref[0]
    acc = jnp.concatenate([a[0], a[1]], axis=-1)
    hv = hp_ref[0]
    hp = jnp.concatenate([hv[0], hv[1]], axis=-1)
    cnt = deg_ref[0][:, :1]
    dis = lax.rsqrt(cnt + 1.0)
    pre = dis * (acc + hp) + b2_ref[0, 0]
    z = _ln_relu(pre, g2_ref[0, 0], be2_ref[0, 0])
    bat = bat_ref[0, 0]
    oh = (bat[:, None] == lax.broadcasted_iota(jnp.int32, (BN, G), 1)
          ).astype(F32)
    pp = lax.dot_general(oh, z, (((0,), (0,)), ((), ())),
                         preferred_element_type=F32)
    cc = jnp.sum(oh, axis=0)
    i = pl.program_id(1)

    @pl.when(i == 0)
    def _():
        pool_ref[0] = jnp.zeros((G, HID), F32)
        cnt_ref[0] = jnp.zeros((G, 128), F32)

    pool_ref[0] += pp
    cnt_ref[0] += cc[:, None]


def _tcd_body(pool_ref, cnt_ref, w1_ref, b1_ref, w2_ref, b2_ref, out_ref):
    c1 = jnp.maximum(cnt_ref[0][:, :1], 1.0)
    c2 = jnp.maximum(cnt_ref[1][:, :1], 1.0)
    p = jnp.concatenate([pool_ref[0] / c1, pool_ref[1] / c2], axis=-1)
    y = jnp.maximum(
        jnp.dot(p, w1_ref[...], preferred_element_type=F32) + b1_ref[0], 0.0)
    out_ref[...] = jnp.dot(y, w2_ref[...],
                           preferred_element_type=F32) + b2_ref[0]


def kernel(n1_x, n1_edge_index, n1_batch, n2_x, n2_edge_index, n2_batch,
           W_n1c1, b_n1c1, g_n1l1, be_n1l1, W_n1c2, b_n1c2, g_n1l2, be_n1l2,
           W_n2c1, b_n2c1, g_n2l1, be_n2l1, W_n2c2, b_n2c2, g_n2l2, be_n2l2,
           W_lin1, b_lin1, W_lin2, b_lin2):
    xs = jnp.pad(jnp.stack([n1_x, n2_x]), ((0, 0), (0, NP - N), (0, 0)))
    W1s = jnp.stack([W_n1c1, W_n2c1])
    W2s = jnp.stack([W_n1c2, W_n2c2])
    b1s = jnp.stack([b_n1c1, b_n2c1]).reshape(2, 1, HID)
    g1s = jnp.stack([g_n1l1, g_n2l1]).reshape(2, 1, HID)
    be1s = jnp.stack([be_n1l1, be_n2l1]).reshape(2, 1, HID)
    b2s = jnp.stack([b_n1c2, b_n2c2]).reshape(2, 1, HID)
    g2s = jnp.stack([g_n1l2, g_n2l2]).reshape(2, 1, HID)
    be2s = jnp.stack([be_n1l2, be_n2l2]).reshape(2, 1, HID)
    batp = jnp.pad(jnp.stack([n1_batch, n2_batch]), ((0, 0), (0, NP - N)),
                   constant_values=G).reshape(2, 1, NP)
    srcs = jnp.stack([n1_edge_index[0], n2_edge_index[0]])
    dsts = jnp.stack([n1_edge_index[1], n2_edge_index[1]])
    srcp = jnp.pad(srcs, ((0, 0), (0, EP - E)), constant_values=N)
    dstp = jnp.pad(dsts, ((0, 0), (0, EP - E)), constant_values=N)
    offs = jnp.array([[0, 2], [1, 3]], jnp.int32) * NP
    src_idx = (srcp[None, :, :] + offs[:, :, None]).reshape(4 * EP // 128, 128)
    dst_idx = dstp.reshape(2 * EP // 128, 128)
    zrow = jnp.zeros((NP, 128), F32)
    zdeg = jnp.zeros((NP, 16), F32)
    e0 = jnp.zeros((128, 16), F32).at[:, 0].set(1.0)

    degs = _deg_kernel(dst_idx, zdeg, e0).reshape(2, NP, 16)

    hp1 = pl.pallas_call(
        _tca_body,
        grid=(2, NP // BN),
        in_specs=[
            pl.BlockSpec((1, BN, FEAT), lambda g, i: (g, i, 0)),
            pl.BlockSpec((1, FEAT, HID), lambda g, i: (g, 0, 0)),
            pl.BlockSpec((1, BN, 16), lambda g, i: (g, i, 0)),
        ],
        out_specs=pl.BlockSpec((1, 2, BN, 128), lambda g, i: (g, 0, i, 0)),
        out_shape=jax.ShapeDtypeStruct((2, 2, NP, 128), F32),
    )(xs, W1s, degs)

    agg1 = _agg_kernel(hp1.reshape(4 * NP, 128), src_idx, dst_idx,
                       zrow).reshape(2, 2, NP, 128)

    hp2 = pl.pallas_call(
        _tcb_body,
        grid=(2, NP // BN),
        in_specs=[
            pl.BlockSpec((1, 2, BN, 128), lambda g, i: (g, 0, i, 0)),
            pl.BlockSpec((1, 2, BN, 128), lambda g, i: (g, 0, i, 0)),
            pl.BlockSpec((1, BN, 16), lambda g, i: (g, i, 0)),
            pl.BlockSpec((1, HID, HID), lambda g, i: (g, 0, 0)),
            pl.BlockSpec((1, 1, HID), lambda g, i: (g, 0, 0)),
            pl.BlockSpec((1, 1, HID), lambda g, i: (g, 0, 0)),
            pl.BlockSpec((1, 1, HID), lambda g, i: (g, 0, 0)),
        ],
        out_specs=pl.BlockSpec((1, 2, BN, 128), lambda g, i: (g, 0, i, 0)),
        out_shape=jax.ShapeDtypeStruct((2, 2, NP, 128), F32),
    )(agg1, hp1, degs, W2s, b1s, g1s, be1s)

    agg2 = _agg_kernel(hp2.reshape(4 * NP, 128), src_idx, dst_idx,
                       zrow).reshape(2, 2, NP, 128)

    pool, cntb = pl.pallas_call(
        _tcc_body,
        grid=(2, NP // BN),
        in_specs=[
            pl.BlockSpec((1, 2, BN, 128), lambda g, i: (g, 0, i, 0)),
            pl.BlockSpec((1, 2, BN, 128), lambda g, i: (g, 0, i, 0)),
            pl.BlockSpec((1, BN, 16), lambda g, i: (g, i, 0)),
            pl.BlockSpec((1, 1, HID), lambda g, i: (g, 0, 0)),
            pl.BlockSpec((1, 1, HID), lambda g, i: (g, 0, 0)),
            pl.BlockSpec((1, 1, HID), lambda g, i: (g, 0, 0)),
            pl.BlockSpec((1, 1, BN), lambda g, i: (g, 0, i)),
        ],
        out_specs=[
            pl.BlockSpec((1, G, HID), lambda g, i: (g, 0, 0)),
            pl.BlockSpec((1, G, 128), lambda g, i: (g, 0, 0)),
        ],
        out_shape=[
            jax.ShapeDtypeStruct((2, G, HID), F32),
            jax.ShapeDtypeStruct((2, G, 128), F32),
        ],
    )(agg2, hp2, degs, b2s, g2s, be2s, batp)

    out = pl.pallas_call(
        _tcd_body,
        out_shape=jax.ShapeDtypeStruct((G, CLS), F32),
    )(pool, cntb, W_lin1, b_lin1.reshape(1, 128), W_lin2,
      b_lin2.reshape(1, CLS))
    return out


# speedup vs baseline: 7.6768x; 1.0001x over previous
import functools

import jax
import jax.numpy as jnp
from jax import lax
from jax.experimental import pallas as pl
from jax.experimental.pallas import tpu as pltpu
from jax.experimental.pallas import tpu_sc as plsc

N = 10000
E = 320000
FEAT = 128
HID = 256
G = 16
CLS = 10

NP = 10240
EP = 327680
ROWS_PER_TILE = EP // 128 // 16
NPT = NP // 16
F32 = jnp.float32

_mesh = plsc.VectorSubcoreMesh(
    core_axis_name="c", subcore_axis_name="s", num_cores=2, num_subcores=16)


@functools.partial(
    pl.kernel,
    out_type=jax.ShapeDtypeStruct((2 * NP, 16), F32),
    mesh=_mesh,
    scratch_types=[
        pltpu.VMEM((16, 128), jnp.int32),
        pltpu.VMEM((128, 16), F32),
        pltpu.VMEM_SHARED((NP, 16), F32),
    ],
)
def _deg_kernel(dst_idx_hbm, zdeg_hbm, e0_hbm, deg_out, didx_v, e0_v, acc_sh):
    cid = lax.axis_index("c")
    sid = lax.axis_index("s")
    pltpu.sync_copy(zdeg_hbm.at[pl.ds(sid * NPT, NPT)],
                    acc_sh.at[pl.ds(sid * NPT, NPT)])
    pltpu.sync_copy(e0_hbm, e0_v)
    plsc.subcore_barrier()
    base = cid * (EP // 128) + sid * ROWS_PER_TILE

    def stage(ch, carry):
        pltpu.sync_copy(dst_idx_hbm.at[pl.ds(base + ch * 16, 16)], didx_v)
        for j in range(16):
            pltpu.sync_copy(e0_v, acc_sh.at[didx_v.at[j]], add=True)
        return carry

    lax.fori_loop(0, ROWS_PER_TILE // 16, stage, 0)
    plsc.subcore_barrier()
    pltpu.sync_copy(acc_sh.at[pl.ds(sid * NPT, NPT)],
                    deg_out.at[pl.ds(cid * NP + sid * NPT, NPT)])


NBUF = 2
LAG = 1


@functools.partial(
    pl.kernel,
    out_type=jax.ShapeDtypeStruct((4 * NP, 128), F32),
    mesh=_mesh,
    scratch_types=[
        pltpu.VMEM((16, 128), jnp.int32),
        pltpu.VMEM((16, 128), jnp.int32),
        pltpu.VMEM((NBUF, 128, 128), F32),
    ]
    + [pltpu.SemaphoreType.DMA] * (2 * NBUF)
    + [pltpu.VMEM_SHARED((NP, 128), F32)],
)
def _agg_kernel(hp_hbm, src_idx_hbm, dst_idx_hbm, zrow_hbm, out_hbm,
                sidx_v, didx_v, rows_v, *rest):
    gsem = rest[:NBUF]
    ssem = rest[NBUF:2 * NBUF]
    acc_sh = rest[2 * NBUF]
    cid = lax.axis_index("c")
    sid = lax.axis_index("s")
    for g in range(2):
        pltpu.sync_copy(zrow_hbm.at[pl.ds(sid * NPT, NPT)],
                        acc_sh.at[pl.ds(sid * NPT, NPT)])
        plsc.subcore_barrier()
        sbase = (cid * 2 + g) * (EP // 128) + sid * ROWS_PER_TILE
        dbase = g * (EP // 128) + sid * ROWS_PER_TILE

        def stage(ch, carry):
            pltpu.sync_copy(src_idx_hbm.at[pl.ds(sbase + ch * 16, 16)], sidx_v)
            pltpu.sync_copy(dst_idx_hbm.at[pl.ds(dbase + ch * 16, 16)], didx_v)
            gd = [None] * NBUF
            for j in range(16 + LAG):
                if j < 16:
                    b = j % NBUF
                    gd[b] = pltpu.async_copy(
                        hp_hbm.at[sidx_v.at[j]], rows_v.at[b], gsem[b])
                i = j - LAG
                if i >= 0:
                    bi = i % NBUF
                    gd[bi].wait()
                    pltpu.sync_copy(rows_v.at[bi], acc_sh.at[didx_v.at[i]],
                                    add=True)
            return carry

        lax.fori_loop(0, ROWS_PER_TILE // 16, stage, 0)
        plsc.subcore_barrier()
        ob = (2 * g + cid) * NP + sid * NPT
        pltpu.sync_copy(acc_sh.at[pl.ds(sid * NPT, NPT)],
                        out_hbm.at[pl.ds(ob, NPT)])


BN = 1024


def _tca_body(x_ref, w_ref, deg_ref, out_ref):
    x = x_ref[0]
    w = w_ref[0]
    cnt = deg_ref[0][:, :1]
    dis = lax.rsqrt(cnt + 1.0)
    h = jnp.dot(x, w, preferred_element_type=F32) * dis
    out_ref[0, 0] = h[:, :128]
    out_ref[0, 1] = h[:, 128:]


def _ln_relu(pre, gam, bet):
    mu = jnp.mean(pre, axis=-1, keepdims=True)
    var = jnp.mean((pre - mu) * (pre - mu), axis=-1, keepdims=True)
    return jnp.maximum((pre - mu) * lax.rsqrt(var + 1e-5) * gam + bet, 0.0)


def _tcb_body(agg_ref, hp_ref, deg_ref, w2_ref, b1_ref, g1_ref, be1_ref,
              out_ref):
    a = agg_ref[0]
    acc = jnp.concatenate([a[0], a[1]], axis=-1)
    hv = hp_ref[0]
    hp = jnp.concatenate([hv[0], hv[1]], axis=-1)
    cnt = deg_ref[0][:, :1]
    dis = lax.rsqrt(cnt + 1.0)
    pre = dis * (acc + hp) + b1_ref[0, 0]
    z = _ln_relu(pre, g1_ref[0, 0], be1_ref[0, 0])
    h2 = jnp.dot(z, w2_ref[0], preferred_element_type=F32) * dis
    out_ref[0, 0] = h2[:, :128]
    out_ref[0, 1] = h2[:, 128:]


def _tcc_body(agg_ref, hp_ref, deg_ref, b2_ref, g2_ref, be2_ref, bat_ref,
              pool_ref, cnt_ref):
    a = agg_ref[0]
    acc = jnp.concatenate([a[0], a[1]], axis=-1)
    hv = hp_ref[0]
    hp = jnp.concatenate([hv[0], hv[1]], axis=-1)
    cnt = deg_ref[0][:, :1]
    dis = lax.rsqrt(cnt + 1.0)
    pre = dis * (acc + hp) + b2_ref[0, 0]
    z = _ln_relu(pre, g2_ref[0, 0], be2_ref[0, 0])
    bat = bat_ref[0, 0]
    oh = (bat[:, None] == lax.broadcasted_iota(jnp.int32, (BN, G), 1)
          ).astype(F32)
    pp = lax.dot_general(oh, z, (((0,), (0,)), ((), ())),
                         preferred_element_type=F32)
    cc = jnp.sum(oh, axis=0)
    i = pl.program_id(1)

    @pl.when(i == 0)
    def _():
        pool_ref[0] = jnp.zeros((G, HID), F32)
        cnt_ref[0] = jnp.zeros((G, 128), F32)

    pool_ref[0] += pp
    cnt_ref[0] += cc[:, None]


def _tcd_body(pool_ref, cnt_ref, w1_ref, b1_ref, w2_ref, b2_ref, out_ref):
    c1 = jnp.maximum(cnt_ref[0][:, :1], 1.0)
    c2 = jnp.maximum(cnt_ref[1][:, :1], 1.0)
    p = jnp.concatenate([pool_ref[0] / c1, pool_ref[1] / c2], axis=-1)
    y = jnp.maximum(
        jnp.dot(p, w1_ref[...], preferred_element_type=F32) + b1_ref[0], 0.0)
    out_ref[...] = jnp.dot(y, w2_ref[...],
                           preferred_element_type=F32) + b2_ref[0]


def kernel(n1_x, n1_edge_index, n1_batch, n2_x, n2_edge_index, n2_batch,
           W_n1c1, b_n1c1, g_n1l1, be_n1l1, W_n1c2, b_n1c2, g_n1l2, be_n1l2,
           W_n2c1, b_n2c1, g_n2l1, be_n2l1, W_n2c2, b_n2c2, g_n2l2, be_n2l2,
           W_lin1, b_lin1, W_lin2, b_lin2):
    xs = jnp.pad(jnp.stack([n1_x, n2_x]), ((0, 0), (0, NP - N), (0, 0)))
    W1s = jnp.stack([W_n1c1, W_n2c1])
    W2s = jnp.stack([W_n1c2, W_n2c2])
    b1s = jnp.stack([b_n1c1, b_n2c1]).reshape(2, 1, HID)
    g1s = jnp.stack([g_n1l1, g_n2l1]).reshape(2, 1, HID)
    be1s = jnp.stack([be_n1l1, be_n2l1]).reshape(2, 1, HID)
    b2s = jnp.stack([b_n1c2, b_n2c2]).reshape(2, 1, HID)
    g2s = jnp.stack([g_n1l2, g_n2l2]).reshape(2, 1, HID)
    be2s = jnp.stack([be_n1l2, be_n2l2]).reshape(2, 1, HID)
    batp = jnp.pad(jnp.stack([n1_batch, n2_batch]), ((0, 0), (0, NP - N)),
                   constant_values=G).reshape(2, 1, NP)
    srcs = jnp.stack([n1_edge_index[0], n2_edge_index[0]])
    dsts = jnp.stack([n1_edge_index[1], n2_edge_index[1]])
    srcp = jnp.pad(srcs, ((0, 0), (0, EP - E)), constant_values=N)
    dstp = jnp.pad(dsts, ((0, 0), (0, EP - E)), constant_values=N)
    offs = jnp.array([[0, 2], [1, 3]], jnp.int32) * NP
    src_idx = (srcp[None, :, :] + offs[:, :, None]).reshape(4 * EP // 128, 128)
    dst_idx = dstp.reshape(2 * EP // 128, 128)
    zrow = jnp.zeros((NP, 128), F32)
    zdeg = jnp.zeros((NP, 16), F32)
    e0 = jnp.zeros((128, 16), F32).at[:, 0].set(1.0)

    degs = _deg_kernel(dst_idx, zdeg, e0).reshape(2, NP, 16)

    hp1 = pl.pallas_call(
        _tca_body,
        grid=(2, NP // BN),
        in_specs=[
            pl.BlockSpec((1, BN, FEAT), lambda g, i: (g, i, 0)),
            pl.BlockSpec((1, FEAT, HID), lambda g, i: (g, 0, 0)),
            pl.BlockSpec((1, BN, 16), lambda g, i: (g, i, 0)),
        ],
        out_specs=pl.BlockSpec((1, 2, BN, 128), lambda g, i: (g, 0, i, 0)),
        out_shape=jax.ShapeDtypeStruct((2, 2, NP, 128), F32),
    )(xs, W1s, degs)

    agg1 = _agg_kernel(hp1.reshape(4 * NP, 128), src_idx, dst_idx,
                       zrow).reshape(2, 2, NP, 128)

    hp2 = pl.pallas_call(
        _tcb_body,
        grid=(2, NP // BN),
        in_specs=[
            pl.BlockSpec((1, 2, BN, 128), lambda g, i: (g, 0, i, 0)),
            pl.BlockSpec((1, 2, BN, 128), lambda g, i: (g, 0, i, 0)),
            pl.BlockSpec((1, BN, 16), lambda g, i: (g, i, 0)),
            pl.BlockSpec((1, HID, HID), lambda g, i: (g, 0, 0)),
            pl.BlockSpec((1, 1, HID), lambda g, i: (g, 0, 0)),
            pl.BlockSpec((1, 1, HID), lambda g, i: (g, 0, 0)),
            pl.BlockSpec((1, 1, HID), lambda g, i: (g, 0, 0)),
        ],
        out_specs=pl.BlockSpec((1, 2, BN, 128), lambda g, i: (g, 0, i, 0)),
        out_shape=jax.ShapeDtypeStruct((2, 2, NP, 128), F32),
    )(agg1, hp1, degs, W2s, b1s, g1s, be1s)

    agg2 = _agg_kernel(hp2.reshape(4 * NP, 128), src_idx, dst_idx,
                       zrow).reshape(2, 2, NP, 128)

    pool, cntb = pl.pallas_call(
        _tcc_body,
        grid=(2, NP // BN),
        in_specs=[
            pl.BlockSpec((1, 2, BN, 128), lambda g, i: (g, 0, i, 0)),
            pl.BlockSpec((1, 2, BN, 128), lambda g, i: (g, 0, i, 0)),
            pl.BlockSpec((1, BN, 16), lambda g, i: (g, i, 0)),
            pl.BlockSpec((1, 1, HID), lambda g, i: (g, 0, 0)),
            pl.BlockSpec((1, 1, HID), lambda g, i: (g, 0, 0)),
            pl.BlockSpec((1, 1, HID), lambda g, i: (g, 0, 0)),
            pl.BlockSpec((1, 1, BN), lambda g, i: (g, 0, i)),
        ],
        out_specs=[
            pl.BlockSpec((1, G, HID), lambda g, i: (g, 0, 0)),
            pl.BlockSpec((1, G, 128), lambda g, i: (g, 0, 0)),
        ],
        out_shape=[
            jax.ShapeDtypeStruct((2, G, HID), F32),
            jax.ShapeDtypeStruct((2, G, 128), F32),
        ],
    )(agg2, hp2, degs, b2s, g2s, be2s, batp)

    out = pl.pallas_call(
        _tcd_body,
        out_shape=jax.ShapeDtypeStruct((G, CLS), F32),
    )(pool, cntb, W_lin1, b_lin1.reshape(1, 128), W_lin2,
      b_lin2.reshape(1, CLS))
    return out
